# Initial kernel scaffold; baseline (speedup 1.0000x reference)
import jax
import jax.numpy as jnp
from jax.experimental import pallas as pl


def kernel(obs, W1, b1, g1, be1, W2, b2, g2, be2, Wa1, ba1, Wa2, ba2, Wa3, ba3, Wih_f, Whh_f, bih_f, bhh_f, Wih_b, Whh_b, bih_b, bhh_b, W3, b3, g3, be3, W4, b4, g4, be4):
    raise NotImplementedError("write your pallas kernel here")


# speedup vs baseline: 9.6046x; 9.6046x over previous
import functools

import jax
import jax.numpy as jnp
from jax.experimental import pallas as pl
from jax.experimental.pallas import tpu as pltpu

B = 2
N = 32
OBS = 256
TH = 256
ACT = 64
M = 8
ATT = 128
H = 128
G4 = 4 * H


def _ln(x, g, b):
    m = jnp.mean(x, axis=-1, keepdims=True)
    v = jnp.var(x, axis=-1, keepdims=True)
    return (x - m) / jnp.sqrt(v + 1e-5) * g + b


def _dot(a, b):
    return jnp.dot(a, b, preferred_element_type=jnp.float32)


def _fused_kernel(obs_ref, W1_ref, b1_ref, g1_ref, be1_ref, W2_ref, b2_ref,
                  g2_ref, be2_ref, Wa1_ref, ba1_ref, Wa2_ref, ba2_ref,
                  Wa3_ref, ba3_ref, WihC_ref, WhhB_ref, biasC_ref,
                  W3_ref, b3_ref, g3_ref, be3_ref, W4_ref, b4_ref, g4_ref,
                  be4_ref, acts_ref, C_ref, SEL_ref, POS_ref, INIT_ref):
    f32 = jnp.float32
    x = obs_ref[...]

    th = _ln(_dot(x, W1_ref[...]) + b1_ref[...], g1_ref[...], be1_ref[...])
    th = jnp.maximum(th, 0.0)
    th = _ln(_dot(th, W2_ref[...]) + b2_ref[...], g2_ref[...], be2_ref[...])

    a = jnp.maximum(_dot(th, Wa1_ref[...]) + ba1_ref[...], 0.0)
    a = jnp.maximum(_dot(a, Wa2_ref[...]) + ba2_ref[...], 0.0)
    z = _dot(a, Wa3_ref[...]) + ba3_ref[...]
    p = jax.nn.sigmoid(z)
    init = (p > 0.5).astype(f32)

    i32 = jnp.int32
    iota_j = jax.lax.broadcasted_iota(i32, (1, N, N), 1)
    iota_k = jax.lax.broadcasted_iota(i32, (1, N, N), 2)
    sels = []
    poss = []
    r_i = jax.lax.broadcasted_iota(i32, (N, N), 0)
    c_i = jax.lax.broadcasted_iota(i32, (N, N), 1)
    UT = (r_i < c_i).astype(f32)
    for b in range(B):
        ob = x[b * N:(b + 1) * N, :]
        diff = ob[:, None, :] - ob[None, :, :]
        d = jnp.sqrt(jnp.sum(diff * diff, axis=-1))
        dj = d[:, :, None]
        dk = d[:, None, :]
        less = dk < dj
        tie = (dk == dj) & (iota_k < iota_j)
        rank = jnp.sum((less | tie).astype(f32), axis=-1)
        sel = (rank < float(M)).astype(f32)
        pos = _dot(sel, UT)
        sels.append(sel)
        poss.append(pos)
    SEL = jnp.concatenate(sels, axis=0)
    POS = jnp.concatenate(poss, axis=0)
    C_ref[...] = SEL * init
    SEL_ref[...] = SEL
    POS_ref[...] = POS
    INIT_ref[...] = init

    WihC = WihC_ref[...]
    WhhB = WhhB_ref[...]
    biasC = biasC_ref[...]
    iota8 = jax.lax.broadcasted_iota(jnp.int32, (M, N), 0).astype(f32)
    z8 = jnp.zeros((M, N), f32)
    ones8 = jnp.ones((M, 1), f32)

    def step(i, th_c):
        s0 = SEL_ref[pl.ds(i, 1), :]
        s1 = SEL_ref[pl.ds(N + i, 1), :]
        p0 = POS_ref[pl.ds(i, 1), :]
        p1 = POS_ref[pl.ds(N + i, 1), :]
        i0 = INIT_ref[pl.ds(i, 1), :]
        i1 = INIT_ref[pl.ds(N + i, 1), :]
        P0 = jnp.where((p0 == iota8) & (s0 > 0.0), 1.0, 0.0)
        P1 = jnp.where((p1 == iota8) & (s1 > 0.0), 1.0, 0.0)
        P2 = jnp.concatenate(
            [jnp.concatenate([P0, z8], axis=1),
             jnp.concatenate([z8, P1], axis=1)], axis=0)

        seq = _dot(P2, th_c)
        xw = _dot(seq, WihC) + biasC

        h = jnp.zeros((B, 2 * H), f32)
        c_f = jnp.zeros((B, H), f32)
        c_b = jnp.zeros((B, H), f32)
        hf = [None] * M
        hb = [None] * M
        for t in range(M):
            xf_t = jnp.concatenate(
                [xw[t:t + 1, 0:G4], xw[M + t:M + t + 1, 0:G4]], axis=0)
            xb_t = jnp.concatenate(
                [xw[M - 1 - t:M - t, G4:2 * G4],
                 xw[2 * M - 1 - t:2 * M - t, G4:2 * G4]], axis=0)
            hW = _dot(h, WhhB)
            gf = xf_t + hW[:, 0:G4]
            gb = xb_t + hW[:, G4:2 * G4]
            i_f = jax.nn.sigmoid(gf[:, 0:H])
            f_f = jax.nn.sigmoid(gf[:, H:2 * H])
            g_f = jnp.tanh(gf[:, 2 * H:3 * H])
            o_f = jax.nn.sigmoid(gf[:, 3 * H:4 * H])
            c_f = f_f * c_f + i_f * g_f
            h_f = o_f * jnp.tanh(c_f)
            i_b = jax.nn.sigmoid(gb[:, 0:H])
            f_b = jax.nn.sigmoid(gb[:, H:2 * H])
            g_b = jnp.tanh(gb[:, 2 * H:3 * H])
            o_b = jax.nn.sigmoid(gb[:, 3 * H:4 * H])
            c_b = f_b * c_b + i_b * g_b
            h_b = o_b * jnp.tanh(c_b)
            h = jnp.concatenate([h_f, h_b], axis=1)
            hf[t] = h_f
            hb[M - 1 - t] = h_b

        integ0 = jnp.concatenate(
            [jnp.concatenate([hf[t][0:1, :] for t in range(M)], axis=0),
             jnp.concatenate([hb[t][0:1, :] for t in range(M)], axis=0)],
            axis=1)
        integ1 = jnp.concatenate(
            [jnp.concatenate([hf[t][1:2, :] for t in range(M)], axis=0),
             jnp.concatenate([hb[t][1:2, :] for t in range(M)], axis=0)],
            axis=1)
        integ = jnp.concatenate([integ0, integ1], axis=0)

        dn = (((0,), (0,)), ((), ()))
        scat = jax.lax.dot_general(P2, integ, dn,
                                   preferred_element_type=f32)
        v = jnp.concatenate([ones8 * i0, ones8 * i1], axis=0)
        mcol = jax.lax.dot_general(P2, v, dn,
                                   preferred_element_type=f32)
        return th_c * (1.0 - mcol) + scat * mcol

    th = jax.lax.fori_loop(0, N, step, th)

    y = jnp.maximum(th, 0.0)
    y = _ln(_dot(y, W3_ref[...]) + b3_ref[...], g3_ref[...], be3_ref[...])
    y = _ln(_dot(y, W4_ref[...]) + b4_ref[...], g4_ref[...], be4_ref[...])
    acts_ref[...] = jnp.tanh(y)


@functools.partial(jax.jit, static_argnames=("interpret",))
def _run(obs, W1, b1, g1, be1, W2, b2, g2, be2, Wa1, ba1, Wa2, ba2, Wa3, ba3,
         Wih_f, Whh_f, bih_f, bhh_f, Wih_b, Whh_b, bih_b, bhh_b,
         W3, b3, g3, be3, W4, b4, g4, be4, interpret=False):
    f32 = jnp.float32
    obs2 = obs.reshape(B * N, OBS)
    WihC = jnp.concatenate([Wih_f, Wih_b], axis=1)
    zH = jnp.zeros((H, 4 * H), f32)
    WhhB = jnp.concatenate(
        [jnp.concatenate([Whh_f, zH], axis=1),
         jnp.concatenate([zH, Whh_b], axis=1)], axis=0)
    biasC = jnp.concatenate([bih_f + bhh_f, bih_b + bhh_b])[None, :]

    row = lambda v: v[None, :]
    acts2, C2 = pl.pallas_call(
        _fused_kernel,
        out_shape=(jax.ShapeDtypeStruct((B * N, ACT), f32),
                   jax.ShapeDtypeStruct((B * N, N), f32)),
        scratch_shapes=[pltpu.VMEM((B * N, N), f32),
                        pltpu.VMEM((B * N, N), f32),
                        pltpu.VMEM((B * N, 1), f32)],
        interpret=interpret,
    )(obs2, W1, row(b1), row(g1), row(be1), W2, row(b2), row(g2), row(be2),
      Wa1, row(ba1), Wa2, row(ba2), Wa3, row(ba3), WihC, WhhB, biasC,
      W3, row(b3), row(g3), row(be3), W4, row(b4), row(g4), row(be4))
    return acts2.reshape(B, N, ACT), C2.reshape(B, N, N)


def kernel(obs, W1, b1, g1, be1, W2, b2, g2, be2, Wa1, ba1, Wa2, ba2, Wa3,
           ba3, Wih_f, Whh_f, bih_f, bhh_f, Wih_b, Whh_b, bih_b, bhh_b,
           W3, b3, g3, be3, W4, b4, g4, be4):
    return _run(obs, W1, b1, g1, be1, W2, b2, g2, be2, Wa1, ba1, Wa2, ba2,
                Wa3, ba3, Wih_f, Whh_f, bih_f, bhh_f, Wih_b, Whh_b, bih_b,
                bhh_b, W3, b3, g3, be3, W4, b4, g4, be4)


# speedup vs baseline: 10.8615x; 1.1309x over previous
import functools

import jax
import jax.numpy as jnp
from jax.experimental import pallas as pl
from jax.experimental.pallas import tpu as pltpu

B = 2
N = 32
OBS = 256
TH = 256
ACT = 64
M = 8
ATT = 128
H = 128
G4 = 4 * H


def _ln(x, g, b):
    m = jnp.mean(x, axis=-1, keepdims=True)
    v = jnp.var(x, axis=-1, keepdims=True)
    return (x - m) / jnp.sqrt(v + 1e-5) * g + b


def _dot(a, b):
    return jnp.dot(a, b, preferred_element_type=jnp.float32)


def _fused_kernel(obs_ref, W1_ref, b1_ref, g1_ref, be1_ref, W2_ref, b2_ref,
                  g2_ref, be2_ref, Wa1_ref, ba1_ref, Wa2_ref, ba2_ref,
                  Wa3_ref, ba3_ref, WihC_ref, WhhB_ref, biasC_ref,
                  W3_ref, b3_ref, g3_ref, be3_ref, W4_ref, b4_ref, g4_ref,
                  be4_ref, acts_ref, C_ref, SEL_ref, POS_ref, INIT_ref):
    f32 = jnp.float32
    x = obs_ref[...]

    th = _ln(_dot(x, W1_ref[...]) + b1_ref[...], g1_ref[...], be1_ref[...])
    th = jnp.maximum(th, 0.0)
    th = _ln(_dot(th, W2_ref[...]) + b2_ref[...], g2_ref[...], be2_ref[...])

    a = jnp.maximum(_dot(th, Wa1_ref[...]) + ba1_ref[...], 0.0)
    a = jnp.maximum(_dot(a, Wa2_ref[...]) + ba2_ref[...], 0.0)
    z = _dot(a, Wa3_ref[...]) + ba3_ref[...]
    p = jax.nn.sigmoid(z)
    init = (p > 0.5).astype(f32)

    i32 = jnp.int32
    sels = []
    poss = []
    r_i = jax.lax.broadcasted_iota(i32, (N, N), 0)
    c_i = jax.lax.broadcasted_iota(i32, (N, N), 1)
    UT = (r_i < c_i).astype(f32)
    lane = jax.lax.broadcasted_iota(i32, (N, N), 1).astype(f32)
    for b in range(B):
        ob = x[b * N:(b + 1) * N, :]
        diff = ob[:, None, :] - ob[None, :, :]
        d = jnp.sqrt(jnp.sum(diff * diff, axis=-1))
        sel = jnp.zeros((N, N), f32)
        work = d
        for _ in range(M):
            mn = jnp.min(work, axis=-1, keepdims=True)
            amn = jnp.min(jnp.where(work == mn, lane, 1e9),
                          axis=-1, keepdims=True)
            hot = (lane == amn).astype(f32)
            sel = sel + hot
            work = jnp.where(hot > 0.0, 1e30, work)
        pos = _dot(sel, UT)
        sels.append(sel)
        poss.append(pos)
    SEL = jnp.concatenate(sels, axis=0)
    POS = jnp.concatenate(poss, axis=0)
    C_ref[...] = SEL * init
    SEL_ref[...] = SEL
    POS_ref[...] = POS
    INIT_ref[...] = init

    WihC = WihC_ref[...]
    WhhB = WhhB_ref[...]
    biasC = biasC_ref[...]
    iota8 = jax.lax.broadcasted_iota(jnp.int32, (M, N), 0).astype(f32)
    z8 = jnp.zeros((M, N), f32)
    ones8 = jnp.ones((M, 1), f32)

    def step(i, th_c):
        s0 = SEL_ref[pl.ds(i, 1), :]
        s1 = SEL_ref[pl.ds(N + i, 1), :]
        p0 = POS_ref[pl.ds(i, 1), :]
        p1 = POS_ref[pl.ds(N + i, 1), :]
        i0 = INIT_ref[pl.ds(i, 1), :]
        i1 = INIT_ref[pl.ds(N + i, 1), :]
        P0 = jnp.where((p0 == iota8) & (s0 > 0.0), 1.0, 0.0)
        P1 = jnp.where((p1 == iota8) & (s1 > 0.0), 1.0, 0.0)
        P2 = jnp.concatenate(
            [jnp.concatenate([P0, z8], axis=1),
             jnp.concatenate([z8, P1], axis=1)], axis=0)

        seq = _dot(P2, th_c)
        xw = _dot(seq, WihC) + biasC

        h = jnp.zeros((B, 2 * H), f32)
        c_f = jnp.zeros((B, H), f32)
        c_b = jnp.zeros((B, H), f32)
        hf = [None] * M
        hb = [None] * M
        for t in range(M):
            xf_t = jnp.concatenate(
                [xw[t:t + 1, 0:G4], xw[M + t:M + t + 1, 0:G4]], axis=0)
            xb_t = jnp.concatenate(
                [xw[M - 1 - t:M - t, G4:2 * G4],
                 xw[2 * M - 1 - t:2 * M - t, G4:2 * G4]], axis=0)
            hW = _dot(h, WhhB)
            gf = xf_t + hW[:, 0:G4]
            gb = xb_t + hW[:, G4:2 * G4]
            i_f = jax.nn.sigmoid(gf[:, 0:H])
            f_f = jax.nn.sigmoid(gf[:, H:2 * H])
            g_f = jnp.tanh(gf[:, 2 * H:3 * H])
            o_f = jax.nn.sigmoid(gf[:, 3 * H:4 * H])
            c_f = f_f * c_f + i_f * g_f
            h_f = o_f * jnp.tanh(c_f)
            i_b = jax.nn.sigmoid(gb[:, 0:H])
            f_b = jax.nn.sigmoid(gb[:, H:2 * H])
            g_b = jnp.tanh(gb[:, 2 * H:3 * H])
            o_b = jax.nn.sigmoid(gb[:, 3 * H:4 * H])
            c_b = f_b * c_b + i_b * g_b
            h_b = o_b * jnp.tanh(c_b)
            h = jnp.concatenate([h_f, h_b], axis=1)
            hf[t] = h_f
            hb[M - 1 - t] = h_b

        integ0 = jnp.concatenate(
            [jnp.concatenate([hf[t][0:1, :] for t in range(M)], axis=0),
             jnp.concatenate([hb[t][0:1, :] for t in range(M)], axis=0)],
            axis=1)
        integ1 = jnp.concatenate(
            [jnp.concatenate([hf[t][1:2, :] for t in range(M)], axis=0),
             jnp.concatenate([hb[t][1:2, :] for t in range(M)], axis=0)],
            axis=1)
        integ = jnp.concatenate([integ0, integ1], axis=0)

        dn = (((0,), (0,)), ((), ()))
        scat = jax.lax.dot_general(P2, integ, dn,
                                   preferred_element_type=f32)
        v = jnp.concatenate([ones8 * i0, ones8 * i1], axis=0)
        mcol = jax.lax.dot_general(P2, v, dn,
                                   preferred_element_type=f32)
        return th_c * (1.0 - mcol) + scat * mcol

    th = jax.lax.fori_loop(0, N, step, th)

    y = jnp.maximum(th, 0.0)
    y = _ln(_dot(y, W3_ref[...]) + b3_ref[...], g3_ref[...], be3_ref[...])
    y = _ln(_dot(y, W4_ref[...]) + b4_ref[...], g4_ref[...], be4_ref[...])
    acts_ref[...] = jnp.tanh(y)


@functools.partial(jax.jit, static_argnames=("interpret",))
def _run(obs, W1, b1, g1, be1, W2, b2, g2, be2, Wa1, ba1, Wa2, ba2, Wa3, ba3,
         Wih_f, Whh_f, bih_f, bhh_f, Wih_b, Whh_b, bih_b, bhh_b,
         W3, b3, g3, be3, W4, b4, g4, be4, interpret=False):
    f32 = jnp.float32
    obs2 = obs.reshape(B * N, OBS)
    WihC = jnp.concatenate([Wih_f, Wih_b], axis=1)
    zH = jnp.zeros((H, 4 * H), f32)
    WhhB = jnp.concatenate(
        [jnp.concatenate([Whh_f, zH], axis=1),
         jnp.concatenate([zH, Whh_b], axis=1)], axis=0)
    biasC = jnp.concatenate([bih_f + bhh_f, bih_b + bhh_b])[None, :]

    row = lambda v: v[None, :]
    acts2, C2 = pl.pallas_call(
        _fused_kernel,
        out_shape=(jax.ShapeDtypeStruct((B * N, ACT), f32),
                   jax.ShapeDtypeStruct((B * N, N), f32)),
        scratch_shapes=[pltpu.VMEM((B * N, N), f32),
                        pltpu.VMEM((B * N, N), f32),
                        pltpu.VMEM((B * N, 1), f32)],
        interpret=interpret,
    )(obs2, W1, row(b1), row(g1), row(be1), W2, row(b2), row(g2), row(be2),
      Wa1, row(ba1), Wa2, row(ba2), Wa3, row(ba3), WihC, WhhB, biasC,
      W3, row(b3), row(g3), row(be3), W4, row(b4), row(g4), row(be4))
    return acts2.reshape(B, N, ACT), C2.reshape(B, N, N)


def kernel(obs, W1, b1, g1, be1, W2, b2, g2, be2, Wa1, ba1, Wa2, ba2, Wa3,
           ba3, Wih_f, Whh_f, bih_f, bhh_f, Wih_b, Whh_b, bih_b, bhh_b,
           W3, b3, g3, be3, W4, b4, g4, be4):
    return _run(obs, W1, b1, g1, be1, W2, b2, g2, be2, Wa1, ba1, Wa2, ba2,
                Wa3, ba3, Wih_f, Whh_f, bih_f, bhh_f, Wih_b, Whh_b, bih_b,
                bhh_b, W3, b3, g3, be3, W4, b4, g4, be4)


# speedup vs baseline: 18.5571x; 1.7085x over previous
import functools

import jax
import jax.numpy as jnp
from jax.experimental import pallas as pl
from jax.experimental.pallas import tpu as pltpu

B = 2
N = 32
OBS = 256
TH = 256
ACT = 64
M = 8
ATT = 128
H = 128
G4 = 4 * H


def _ln(x, g, b):
    m = jnp.mean(x, axis=-1, keepdims=True)
    v = jnp.var(x, axis=-1, keepdims=True)
    return (x - m) / jnp.sqrt(v + 1e-5) * g + b


def _dot(a, b):
    return jnp.dot(a, b, preferred_element_type=jnp.float32)


def _select_kernel(obs_ref, W1_ref, b1_ref, g1_ref, be1_ref, W2_ref, b2_ref,
                   g2_ref, be2_ref, Wa1_ref, ba1_ref, Wa2_ref, ba2_ref,
                   Wa3_ref, ba3_ref, I64_ref,
                   th_ref, C_ref, SELc_ref, POSc_ref, cnt_ref):
    f32 = jnp.float32
    i32 = jnp.int32
    x = obs_ref[...]

    th = _ln(_dot(x, W1_ref[...]) + b1_ref[...], g1_ref[...], be1_ref[...])
    th = jnp.maximum(th, 0.0)
    th = _ln(_dot(th, W2_ref[...]) + b2_ref[...], g2_ref[...], be2_ref[...])
    th_ref[...] = th

    a = jnp.maximum(_dot(th, Wa1_ref[...]) + ba1_ref[...], 0.0)
    a = jnp.maximum(_dot(a, Wa2_ref[...]) + ba2_ref[...], 0.0)
    z = _dot(a, Wa3_ref[...]) + ba3_ref[...]
    p = jax.nn.sigmoid(z)
    init = (p > 0.5).astype(f32)
    irow = jax.lax.dot_general(init, I64_ref[...], (((0,), (0,)), ((), ())),
                               preferred_element_type=f32)

    r_i = jax.lax.broadcasted_iota(i32, (N, N), 0)
    c_i = jax.lax.broadcasted_iota(i32, (N, N), 1)
    UT = (r_i < c_i).astype(f32)
    lane = jax.lax.broadcasted_iota(i32, (N, N), 1).astype(f32)
    row_i = jax.lax.broadcasted_iota(i32, (N, N), 0).astype(f32)
    selc, posc, cnts = [], [], []
    for b in range(B):
        ob = x[b * N:(b + 1) * N, :]
        diff = ob[:, None, :] - ob[None, :, :]
        d = jnp.sqrt(jnp.sum(diff * diff, axis=-1))
        sel = jnp.zeros((N, N), f32)
        work = d
        for _ in range(M):
            mn = jnp.min(work, axis=-1, keepdims=True)
            amn = jnp.min(jnp.where(work == mn, lane, 1e9),
                          axis=-1, keepdims=True)
            hot = (lane == amn).astype(f32)
            sel = sel + hot
            work = jnp.where(hot > 0.0, 1e30, work)
        pos = _dot(sel, UT)
        irow_b = irow[:, b * N:(b + 1) * N]
        C_ref[b * N:(b + 1) * N, :] = sel * init[b * N:(b + 1) * N, :]
        rk = _dot(irow_b, UT)
        G = jnp.where((row_i == rk) & (irow_b > 0.0), 1.0, 0.0)
        selc.append(_dot(G, sel))
        posc.append(_dot(G, pos))
        cnts.append(jnp.sum(irow_b, axis=-1, keepdims=True))
    SELc_ref[...] = jnp.concatenate(selc, axis=0)
    POSc_ref[...] = jnp.concatenate(posc, axis=0)
    cnt_ref[...] = jnp.concatenate(
        cnts + [jnp.zeros((1, 128 - B), f32)], axis=1)


def _integrate_kernel(nmax_ref, th_ref, SELc_ref, POSc_ref, WihC_ref,
                      WhhB_ref, biasC_ref, W3_ref, b3_ref, g3_ref, be3_ref,
                      W4_ref, b4_ref, g4_ref, be4_ref, acts_ref):
    f32 = jnp.float32
    WihC = WihC_ref[...]
    WhhB = WhhB_ref[...]
    biasC = biasC_ref[...]
    iota8 = jax.lax.broadcasted_iota(jnp.int32, (M, N), 0).astype(f32)
    z8 = jnp.zeros((M, N), f32)
    ones16 = jnp.ones((2 * M, 1), f32)

    def step(k, th_c):
        s0 = SELc_ref[pl.ds(k, 1), :]
        s1 = SELc_ref[pl.ds(N + k, 1), :]
        p0 = POSc_ref[pl.ds(k, 1), :]
        p1 = POSc_ref[pl.ds(N + k, 1), :]
        P0 = jnp.where((p0 == iota8) & (s0 > 0.0), 1.0, 0.0)
        P1 = jnp.where((p1 == iota8) & (s1 > 0.0), 1.0, 0.0)
        P2 = jnp.concatenate(
            [jnp.concatenate([P0, z8], axis=1),
             jnp.concatenate([z8, P1], axis=1)], axis=0)

        seq = _dot(P2, th_c)
        xw = _dot(seq, WihC) + biasC

        h = jnp.zeros((B, 2 * H), f32)
        c_f = jnp.zeros((B, H), f32)
        c_b = jnp.zeros((B, H), f32)
        hf = [None] * M
        hb = [None] * M
        for t in range(M):
            xf_t = jnp.concatenate(
                [xw[t:t + 1, 0:G4], xw[M + t:M + t + 1, 0:G4]], axis=0)
            xb_t = jnp.concatenate(
                [xw[M - 1 - t:M - t, G4:2 * G4],
                 xw[2 * M - 1 - t:2 * M - t, G4:2 * G4]], axis=0)
            hW = _dot(h, WhhB)
            gf = xf_t + hW[:, 0:G4]
            gb = xb_t + hW[:, G4:2 * G4]
            i_f = jax.nn.sigmoid(gf[:, 0:H])
            f_f = jax.nn.sigmoid(gf[:, H:2 * H])
            g_f = jnp.tanh(gf[:, 2 * H:3 * H])
            o_f = jax.nn.sigmoid(gf[:, 3 * H:4 * H])
            c_f = f_f * c_f + i_f * g_f
            h_f = o_f * jnp.tanh(c_f)
            i_b = jax.nn.sigmoid(gb[:, 0:H])
            f_b = jax.nn.sigmoid(gb[:, H:2 * H])
            g_b = jnp.tanh(gb[:, 2 * H:3 * H])
            o_b = jax.nn.sigmoid(gb[:, 3 * H:4 * H])
            c_b = f_b * c_b + i_b * g_b
            h_b = o_b * jnp.tanh(c_b)
            h = jnp.concatenate([h_f, h_b], axis=1)
            hf[t] = h_f
            hb[M - 1 - t] = h_b

        integ0 = jnp.concatenate(
            [jnp.concatenate([hf[t][0:1, :] for t in range(M)], axis=0),
             jnp.concatenate([hb[t][0:1, :] for t in range(M)], axis=0)],
            axis=1)
        integ1 = jnp.concatenate(
            [jnp.concatenate([hf[t][1:2, :] for t in range(M)], axis=0),
             jnp.concatenate([hb[t][1:2, :] for t in range(M)], axis=0)],
            axis=1)
        integ = jnp.concatenate([integ0, integ1], axis=0)

        dn = (((0,), (0,)), ((), ()))
        scat = jax.lax.dot_general(P2, integ, dn,
                                   preferred_element_type=f32)
        mcol = jax.lax.dot_general(P2, ones16, dn,
                                   preferred_element_type=f32)
        return th_c * (1.0 - mcol) + scat * mcol

    th = jax.lax.fori_loop(0, nmax_ref[0], step, th_ref[...])

    y = jnp.maximum(th, 0.0)
    y = _ln(_dot(y, W3_ref[...]) + b3_ref[...], g3_ref[...], be3_ref[...])
    y = _ln(_dot(y, W4_ref[...]) + b4_ref[...], g4_ref[...], be4_ref[...])
    acts_ref[...] = jnp.tanh(y)


@functools.partial(jax.jit, static_argnames=("interpret",))
def _run(obs, W1, b1, g1, be1, W2, b2, g2, be2, Wa1, ba1, Wa2, ba2, Wa3, ba3,
         Wih_f, Whh_f, bih_f, bhh_f, Wih_b, Whh_b, bih_b, bhh_b,
         W3, b3, g3, be3, W4, b4, g4, be4, interpret=False):
    f32 = jnp.float32
    obs2 = obs.reshape(B * N, OBS)
    row = lambda v: v[None, :]
    I64 = jnp.eye(B * N, dtype=f32)

    th, C2, SELc, POSc, cnt = pl.pallas_call(
        _select_kernel,
        out_shape=(jax.ShapeDtypeStruct((B * N, TH), f32),
                   jax.ShapeDtypeStruct((B * N, N), f32),
                   jax.ShapeDtypeStruct((B * N, N), f32),
                   jax.ShapeDtypeStruct((B * N, N), f32),
                   jax.ShapeDtypeStruct((1, 128), f32)),
        interpret=interpret,
    )(obs2, W1, row(b1), row(g1), row(be1), W2, row(b2), row(g2), row(be2),
      Wa1, row(ba1), Wa2, row(ba2), Wa3, row(ba3), I64)

    nmax = jnp.maximum(cnt[0, 0], cnt[0, 1]).astype(jnp.int32).reshape(1)

    WihC = jnp.concatenate([Wih_f, Wih_b], axis=1)
    zH = jnp.zeros((H, 4 * H), f32)
    WhhB = jnp.concatenate(
        [jnp.concatenate([Whh_f, zH], axis=1),
         jnp.concatenate([zH, Whh_b], axis=1)], axis=0)
    biasC = jnp.concatenate([bih_f + bhh_f, bih_b + bhh_b])[None, :]

    n_in = 14
    acts2 = pl.pallas_call(
        _integrate_kernel,
        out_shape=jax.ShapeDtypeStruct((B * N, ACT), f32),
        in_specs=[pl.BlockSpec(memory_space=pltpu.SMEM)] +
                 [pl.BlockSpec(memory_space=pltpu.VMEM)] * n_in,
        interpret=interpret,
    )(nmax, th, SELc, POSc, WihC, WhhB, biasC,
      W3, row(b3), row(g3), row(be3), W4, row(b4), row(g4), row(be4))
    return acts2.reshape(B, N, ACT), C2.reshape(B, N, N)


def kernel(obs, W1, b1, g1, be1, W2, b2, g2, be2, Wa1, ba1, Wa2, ba2, Wa3,
           ba3, Wih_f, Whh_f, bih_f, bhh_f, Wih_b, Whh_b, bih_b, bhh_b,
           W3, b3, g3, be3, W4, b4, g4, be4):
    return _run(obs, W1, b1, g1, be1, W2, b2, g2, be2, Wa1, ba1, Wa2, ba2,
                Wa3, ba3, Wih_f, Whh_f, bih_f, bhh_f, Wih_b, Whh_b, bih_b,
                bhh_b, W3, b3, g3, be3, W4, b4, g4, be4)


# speedup vs baseline: 20.9864x; 1.1309x over previous
import functools

import jax
import jax.numpy as jnp
from jax.experimental import pallas as pl
from jax.experimental.pallas import tpu as pltpu

B = 2
N = 32
OBS = 256
TH = 256
ACT = 64
M = 8
ATT = 128
H = 128
G4 = 4 * H


def _ln(x, g, b):
    m = jnp.mean(x, axis=-1, keepdims=True)
    v = jnp.var(x, axis=-1, keepdims=True)
    return (x - m) / jnp.sqrt(v + 1e-5) * g + b


def _dot(a, b):
    return jnp.dot(a, b, preferred_element_type=jnp.float32)


def _select_kernel(obs_ref, W1_ref, b1_ref, g1_ref, be1_ref, W2_ref, b2_ref,
                   g2_ref, be2_ref, Wa1_ref, ba1_ref, Wa2_ref, ba2_ref,
                   Wa3_ref, ba3_ref, I64_ref,
                   th_ref, C_ref, SELc_ref, POSc_ref, cnt_ref):
    f32 = jnp.float32
    i32 = jnp.int32
    x = obs_ref[...]

    th = _ln(_dot(x, W1_ref[...]) + b1_ref[...], g1_ref[...], be1_ref[...])
    th = jnp.maximum(th, 0.0)
    th = _ln(_dot(th, W2_ref[...]) + b2_ref[...], g2_ref[...], be2_ref[...])
    th_ref[...] = th

    a = jnp.maximum(_dot(th, Wa1_ref[...]) + ba1_ref[...], 0.0)
    a = jnp.maximum(_dot(a, Wa2_ref[...]) + ba2_ref[...], 0.0)
    z = _dot(a, Wa3_ref[...]) + ba3_ref[...]
    p = jax.nn.sigmoid(z)
    init = (p > 0.5).astype(f32)
    irow = jax.lax.dot_general(init, I64_ref[...], (((0,), (0,)), ((), ())),
                               preferred_element_type=f32)

    r_i = jax.lax.broadcasted_iota(i32, (N, N), 0)
    c_i = jax.lax.broadcasted_iota(i32, (N, N), 1)
    UT = (r_i < c_i).astype(f32)
    lane = jax.lax.broadcasted_iota(i32, (N, N), 1).astype(f32)
    row_i = jax.lax.broadcasted_iota(i32, (N, N), 0).astype(f32)
    selc, posc, cnts = [], [], []
    for b in range(B):
        ob = x[b * N:(b + 1) * N, :]
        diff = ob[:, None, :] - ob[None, :, :]
        d = jnp.sqrt(jnp.sum(diff * diff, axis=-1))
        sel = jnp.zeros((N, N), f32)
        work = d
        for _ in range(M):
            mn = jnp.min(work, axis=-1, keepdims=True)
            amn = jnp.min(jnp.where(work == mn, lane, 1e9),
                          axis=-1, keepdims=True)
            hot = (lane == amn).astype(f32)
            sel = sel + hot
            work = jnp.where(hot > 0.0, 1e30, work)
        pos = _dot(sel, UT)
        irow_b = irow[:, b * N:(b + 1) * N]
        C_ref[b * N:(b + 1) * N, :] = sel * init[b * N:(b + 1) * N, :]
        rk = _dot(irow_b, UT)
        G = jnp.where((row_i == rk) & (irow_b > 0.0), 1.0, 0.0)
        selc.append(_dot(G, sel))
        posc.append(_dot(G, pos))
        cnts.append(jnp.sum(irow_b, axis=-1, keepdims=True))
    SELc_ref[...] = jnp.concatenate(selc, axis=0)
    POSc_ref[...] = jnp.concatenate(posc, axis=0)
    cnt_ref[...] = jnp.concatenate(
        cnts + [jnp.zeros((1, 128 - B), f32)], axis=1)


def _integrate_kernel(nmax_ref, th_ref, SELc_ref, POSc_ref, WihC_ref,
                      Whhf_ref, Whhb_ref, biasC_ref, W3_ref, b3_ref, g3_ref,
                      be3_ref, W4_ref, b4_ref, g4_ref, be4_ref, acts_ref):
    f32 = jnp.float32
    WihC = WihC_ref[...]
    Whh_f = Whhf_ref[...]
    Whh_b = Whhb_ref[...]
    biasC = biasC_ref[...]
    rpar = (jax.lax.broadcasted_iota(jnp.int32, (2 * M, N), 0)
            // 2).astype(f32)
    even = (jax.lax.broadcasted_iota(jnp.int32, (2 * M, N), 0) % 2) == 0
    ones16 = jnp.ones((2 * M, 1), f32)

    def step(k, th_c):
        s0 = SELc_ref[pl.ds(k, 1), :]
        s1 = SELc_ref[pl.ds(N + k, 1), :]
        p0 = POSc_ref[pl.ds(k, 1), :]
        p1 = POSc_ref[pl.ds(N + k, 1), :]
        left = jnp.where((p0 == rpar) & (s0 > 0.0) & even, 1.0, 0.0)
        right = jnp.where((p1 == rpar) & (s1 > 0.0) & (~even), 1.0, 0.0)
        P2 = jnp.concatenate([left, right], axis=1)

        seq = _dot(P2, th_c)
        xw = _dot(seq, WihC) + biasC

        h_f = jnp.zeros((B, H), f32)
        h_b = jnp.zeros((B, H), f32)
        c_f = jnp.zeros((B, H), f32)
        c_b = jnp.zeros((B, H), f32)
        hf = [None] * M
        hb = [None] * M
        for t in range(M):
            u = M - 1 - t
            gf = (xw[2 * t:2 * t + 2, 0:G4]
                  + _dot(h_f, Whh_f))
            gb = (xw[2 * u:2 * u + 2, G4:2 * G4]
                  + _dot(h_b, Whh_b))
            i_f = jax.nn.sigmoid(gf[:, 0:H])
            f_f = jax.nn.sigmoid(gf[:, H:2 * H])
            g_f = jnp.tanh(gf[:, 2 * H:3 * H])
            o_f = jax.nn.sigmoid(gf[:, 3 * H:4 * H])
            c_f = f_f * c_f + i_f * g_f
            h_f = o_f * jnp.tanh(c_f)
            i_b = jax.nn.sigmoid(gb[:, 0:H])
            f_b = jax.nn.sigmoid(gb[:, H:2 * H])
            g_b = jnp.tanh(gb[:, 2 * H:3 * H])
            o_b = jax.nn.sigmoid(gb[:, 3 * H:4 * H])
            c_b = f_b * c_b + i_b * g_b
            h_b = o_b * jnp.tanh(c_b)
            hf[t] = h_f
            hb[u] = h_b

        integ = jnp.concatenate(
            [jnp.concatenate([hf[t], hb[t]], axis=1) for t in range(M)],
            axis=0)

        dn = (((0,), (0,)), ((), ()))
        scat = jax.lax.dot_general(P2, integ, dn,
                                   preferred_element_type=f32)
        mcol = jax.lax.dot_general(P2, ones16, dn,
                                   preferred_element_type=f32)
        return th_c * (1.0 - mcol) + scat * mcol

    th = jax.lax.fori_loop(0, nmax_ref[0], step, th_ref[...])

    y = jnp.maximum(th, 0.0)
    y = _ln(_dot(y, W3_ref[...]) + b3_ref[...], g3_ref[...], be3_ref[...])
    y = _ln(_dot(y, W4_ref[...]) + b4_ref[...], g4_ref[...], be4_ref[...])
    acts_ref[...] = jnp.tanh(y)


@functools.partial(jax.jit, static_argnames=("interpret",))
def _run(obs, W1, b1, g1, be1, W2, b2, g2, be2, Wa1, ba1, Wa2, ba2, Wa3, ba3,
         Wih_f, Whh_f, bih_f, bhh_f, Wih_b, Whh_b, bih_b, bhh_b,
         W3, b3, g3, be3, W4, b4, g4, be4, interpret=False):
    f32 = jnp.float32
    obs2 = obs.reshape(B * N, OBS)
    row = lambda v: v[None, :]
    I64 = jnp.eye(B * N, dtype=f32)

    th, C2, SELc, POSc, cnt = pl.pallas_call(
        _select_kernel,
        out_shape=(jax.ShapeDtypeStruct((B * N, TH), f32),
                   jax.ShapeDtypeStruct((B * N, N), f32),
                   jax.ShapeDtypeStruct((B * N, N), f32),
                   jax.ShapeDtypeStruct((B * N, N), f32),
                   jax.ShapeDtypeStruct((1, 128), f32)),
        interpret=interpret,
    )(obs2, W1, row(b1), row(g1), row(be1), W2, row(b2), row(g2), row(be2),
      Wa1, row(ba1), Wa2, row(ba2), Wa3, row(ba3), I64)

    nmax = jnp.maximum(cnt[0, 0], cnt[0, 1]).astype(jnp.int32).reshape(1)

    WihC = jnp.concatenate([Wih_f, Wih_b], axis=1)
    biasC = jnp.concatenate([bih_f + bhh_f, bih_b + bhh_b])[None, :]

    n_in = 15
    acts2 = pl.pallas_call(
        _integrate_kernel,
        out_shape=jax.ShapeDtypeStruct((B * N, ACT), f32),
        in_specs=[pl.BlockSpec(memory_space=pltpu.SMEM)] +
                 [pl.BlockSpec(memory_space=pltpu.VMEM)] * n_in,
        interpret=interpret,
    )(nmax, th, SELc, POSc, WihC, Whh_f, Whh_b, biasC,
      W3, row(b3), row(g3), row(be3), W4, row(b4), row(g4), row(be4))
    return acts2.reshape(B, N, ACT), C2.reshape(B, N, N)


def kernel(obs, W1, b1, g1, be1, W2, b2, g2, be2, Wa1, ba1, Wa2, ba2, Wa3,
           ba3, Wih_f, Whh_f, bih_f, bhh_f, Wih_b, Whh_b, bih_b, bhh_b,
           W3, b3, g3, be3, W4, b4, g4, be4):
    return _run(obs, W1, b1, g1, be1, W2, b2, g2, be2, Wa1, ba1, Wa2, ba2,
                Wa3, ba3, Wih_f, Whh_f, bih_f, bhh_f, Wih_b, Whh_b, bih_b,
                bhh_b, W3, b3, g3, be3, W4, b4, g4, be4)


# speedup vs baseline: 26.3963x; 1.2578x over previous
import functools

import jax
import jax.numpy as jnp
from jax.experimental import pallas as pl
from jax.experimental.pallas import tpu as pltpu

B = 2
N = 32
OBS = 256
TH = 256
ACT = 64
M = 8
ATT = 128
H = 128
G4 = 4 * H


def _ln(x, g, b):
    m = jnp.mean(x, axis=-1, keepdims=True)
    v = jnp.var(x, axis=-1, keepdims=True)
    return (x - m) / jnp.sqrt(v + 1e-5) * g + b


def _dot(a, b):
    return jnp.dot(a, b, preferred_element_type=jnp.float32)


def _fused_kernel(obs_ref, W1_ref, b1_ref, g1_ref, be1_ref, W2_ref, b2_ref,
                  g2_ref, be2_ref, Wa1_ref, ba1_ref, Wa2_ref, ba2_ref,
                  Wa3_ref, ba3_ref, I64_ref, Wihf_ref, Wihb_ref, Whhf_ref,
                  Whhb_ref, bihf_ref, bhhf_ref, bihb_ref, bhhb_ref,
                  W3_ref, b3_ref, g3_ref, be3_ref, W4_ref, b4_ref, g4_ref,
                  be4_ref, acts_ref, C_ref, SELc_ref, POSc_ref, CNT_ref):
    f32 = jnp.float32
    i32 = jnp.int32
    x = obs_ref[...]

    th = _ln(_dot(x, W1_ref[...]) + b1_ref[...], g1_ref[...], be1_ref[...])
    th = jnp.maximum(th, 0.0)
    th = _ln(_dot(th, W2_ref[...]) + b2_ref[...], g2_ref[...], be2_ref[...])

    a = jnp.maximum(_dot(th, Wa1_ref[...]) + ba1_ref[...], 0.0)
    a = jnp.maximum(_dot(a, Wa2_ref[...]) + ba2_ref[...], 0.0)
    z = _dot(a, Wa3_ref[...]) + ba3_ref[...]
    p = jax.nn.sigmoid(z)
    init = (p > 0.5).astype(f32)
    irow = jax.lax.dot_general(init, I64_ref[...], (((0,), (0,)), ((), ())),
                               preferred_element_type=f32)

    r_i = jax.lax.broadcasted_iota(i32, (N, N), 0)
    c_i = jax.lax.broadcasted_iota(i32, (N, N), 1)
    UT = (r_i < c_i).astype(f32)
    lane = jax.lax.broadcasted_iota(i32, (N, N), 1).astype(f32)
    row_i = jax.lax.broadcasted_iota(i32, (N, N), 0).astype(f32)
    selc, posc, cnts = [], [], []
    for b in range(B):
        ob = x[b * N:(b + 1) * N, :]
        diff = ob[:, None, :] - ob[None, :, :]
        d = jnp.sqrt(jnp.sum(diff * diff, axis=-1))
        sel = jnp.zeros((N, N), f32)
        work = d
        for _ in range(M):
            mn = jnp.min(work, axis=-1, keepdims=True)
            amn = jnp.min(jnp.where(work == mn, lane, 1e9),
                          axis=-1, keepdims=True)
            hot = (lane == amn).astype(f32)
            sel = sel + hot
            work = jnp.where(hot > 0.0, 1e30, work)
        pos = _dot(sel, UT)
        irow_b = irow[:, b * N:(b + 1) * N]
        C_ref[b * N:(b + 1) * N, :] = sel * init[b * N:(b + 1) * N, :]
        rk = _dot(irow_b, UT)
        G = jnp.where((row_i == rk) & (irow_b > 0.0), 1.0, 0.0)
        selc.append(_dot(G, sel))
        posc.append(_dot(G, pos))
        cnts.append(jnp.sum(irow_b, axis=-1, keepdims=True))
    SELc_ref[...] = jnp.concatenate(selc, axis=0)
    POSc_ref[...] = jnp.concatenate(posc, axis=0)
    CNT_ref[...] = jnp.concatenate(
        cnts + [jnp.zeros((1, 128 - B), f32)], axis=1)
    nmax = jnp.maximum(CNT_ref[0, 0], CNT_ref[0, 1]).astype(i32)

    Wih_f = Wihf_ref[...]
    Wih_b = Wihb_ref[...]
    Whh_f = Whhf_ref[...]
    Whh_b = Whhb_ref[...]
    biasf = bihf_ref[...] + bhhf_ref[...]
    biasb = bihb_ref[...] + bhhb_ref[...]
    rpar = (jax.lax.broadcasted_iota(i32, (2 * M, N), 0) // 2).astype(f32)
    even = (jax.lax.broadcasted_iota(i32, (2 * M, N), 0) % 2) == 0
    ones16 = jnp.ones((2 * M, 1), f32)

    def step(k, th_c):
        s0 = SELc_ref[pl.ds(k, 1), :]
        s1 = SELc_ref[pl.ds(N + k, 1), :]
        p0 = POSc_ref[pl.ds(k, 1), :]
        p1 = POSc_ref[pl.ds(N + k, 1), :]
        left = jnp.where((p0 == rpar) & (s0 > 0.0) & even, 1.0, 0.0)
        right = jnp.where((p1 == rpar) & (s1 > 0.0) & (~even), 1.0, 0.0)
        P2 = jnp.concatenate([left, right], axis=1)

        seq = _dot(P2, th_c)
        xwf = _dot(seq, Wih_f) + biasf
        xwb = _dot(seq, Wih_b) + biasb

        h_f = jnp.zeros((B, H), f32)
        h_b = jnp.zeros((B, H), f32)
        c_f = jnp.zeros((B, H), f32)
        c_b = jnp.zeros((B, H), f32)
        hf = [None] * M
        hb = [None] * M
        for t in range(M):
            u = M - 1 - t
            gf = xwf[2 * t:2 * t + 2, :] + _dot(h_f, Whh_f)
            gb = xwb[2 * u:2 * u + 2, :] + _dot(h_b, Whh_b)
            i_f = jax.nn.sigmoid(gf[:, 0:H])
            f_f = jax.nn.sigmoid(gf[:, H:2 * H])
            g_f = jnp.tanh(gf[:, 2 * H:3 * H])
            o_f = jax.nn.sigmoid(gf[:, 3 * H:4 * H])
            c_f = f_f * c_f + i_f * g_f
            h_f = o_f * jnp.tanh(c_f)
            i_b = jax.nn.sigmoid(gb[:, 0:H])
            f_b = jax.nn.sigmoid(gb[:, H:2 * H])
            g_b = jnp.tanh(gb[:, 2 * H:3 * H])
            o_b = jax.nn.sigmoid(gb[:, 3 * H:4 * H])
            c_b = f_b * c_b + i_b * g_b
            h_b = o_b * jnp.tanh(c_b)
            hf[t] = h_f
            hb[u] = h_b

        integ = jnp.concatenate(
            [jnp.concatenate([hf[t], hb[t]], axis=1) for t in range(M)],
            axis=0)

        dn = (((0,), (0,)), ((), ()))
        scat = jax.lax.dot_general(P2, integ, dn,
                                   preferred_element_type=f32)
        mcol = jax.lax.dot_general(P2, ones16, dn,
                                   preferred_element_type=f32)
        return th_c * (1.0 - mcol) + scat * mcol

    th = jax.lax.fori_loop(0, nmax, step, th)

    y = jnp.maximum(th, 0.0)
    y = _ln(_dot(y, W3_ref[...]) + b3_ref[...], g3_ref[...], be3_ref[...])
    y = _ln(_dot(y, W4_ref[...]) + b4_ref[...], g4_ref[...], be4_ref[...])
    acts_ref[...] = jnp.tanh(y)


@functools.partial(jax.jit, static_argnames=("interpret",))
def _run(obs, W1, b1, g1, be1, W2, b2, g2, be2, Wa1, ba1, Wa2, ba2, Wa3, ba3,
         Wih_f, Whh_f, bih_f, bhh_f, Wih_b, Whh_b, bih_b, bhh_b,
         W3, b3, g3, be3, W4, b4, g4, be4, interpret=False):
    f32 = jnp.float32
    obs2 = obs.reshape(B * N, OBS)
    row = lambda v: v[None, :]
    I64 = jnp.eye(B * N, dtype=f32)

    acts2, C2 = pl.pallas_call(
        _fused_kernel,
        out_shape=(jax.ShapeDtypeStruct((B * N, ACT), f32),
                   jax.ShapeDtypeStruct((B * N, N), f32)),
        scratch_shapes=[pltpu.VMEM((B * N, N), f32),
                        pltpu.VMEM((B * N, N), f32),
                        pltpu.VMEM((1, 128), f32)],
        interpret=interpret,
    )(obs2, W1, row(b1), row(g1), row(be1), W2, row(b2), row(g2), row(be2),
      Wa1, row(ba1), Wa2, row(ba2), Wa3, row(ba3), I64,
      Wih_f, Wih_b, Whh_f, Whh_b,
      row(bih_f), row(bhh_f), row(bih_b), row(bhh_b),
      W3, row(b3), row(g3), row(be3), W4, row(b4), row(g4), row(be4))
    return acts2.reshape(B, N, ACT), C2.reshape(B, N, N)


def kernel(obs, W1, b1, g1, be1, W2, b2, g2, be2, Wa1, ba1, Wa2, ba2, Wa3,
           ba3, Wih_f, Whh_f, bih_f, bhh_f, Wih_b, Whh_b, bih_b, bhh_b,
           W3, b3, g3, be3, W4, b4, g4, be4):
    return _run(obs, W1, b1, g1, be1, W2, b2, g2, be2, Wa1, ba1, Wa2, ba2,
                Wa3, ba3, Wih_f, Whh_f, bih_f, bhh_f, Wih_b, Whh_b, bih_b,
                bhh_b, W3, b3, g3, be3, W4, b4, g4, be4)
